# Initial kernel scaffold; baseline (speedup 1.0000x reference)
import jax
import jax.numpy as jnp
from jax.experimental import pallas as pl


def kernel(x, edge_index, W1, b1, W2, b2, W3, b3):
    raise NotImplementedError("write your pallas kernel here")


# speedup vs baseline: 8.7997x; 8.7997x over previous
import functools

import jax
import jax.numpy as jnp
from jax import lax
from jax.experimental import pallas as pl
from jax.experimental.pallas import tpu as pltpu
from jax.experimental.pallas import tpu_sc as plsc

N = 10000
F_IN = 128
HID = 128
C = 64
E = 320000

N_ACC = 10240
NTILES = 32
EPT = 10240
E_PAD = NTILES * EPT
CH = 128
NCH = EPT // CH
RPT = N_ACC // 16
DEG_W = 16

_MESH = dict(core_axis_name="c", subcore_axis_name="s")
_SC_PARAMS = pltpu.CompilerParams(use_tc_tiling_on_sc=False)


def _deg_call(dst_p, ones16, zeros16):
  mesh = plsc.VectorSubcoreMesh(**_MESH)

  @functools.partial(
      pl.kernel,
      mesh=mesh,
      out_type=jax.ShapeDtypeStruct((2, N_ACC, DEG_W), jnp.float32),
      scratch_types=[
          pltpu.VMEM((CH,), jnp.int32),
          pltpu.VMEM((CH, DEG_W), jnp.float32),
          pltpu.VMEM((CH, DEG_W), jnp.float32),
          pltpu.VMEM_SHARED((N_ACC, DEG_W), jnp.float32),
      ],
      compiler_params=_SC_PARAMS,
  )
  def k(dst_hbm, ones_hbm, zeros_hbm, out_hbm, dst_v, ones_v, buf_v, acc):
    cid = lax.axis_index("c")
    sid = lax.axis_index("s")
    wid = sid * 2 + cid
    pltpu.sync_copy(ones_hbm, ones_v)
    pltpu.sync_copy(zeros_hbm, buf_v)
    for j in range(RPT // CH):
      pltpu.sync_copy(buf_v, acc.at[pl.ds(sid * RPT + j * CH, CH)])
    plsc.subcore_barrier()

    def body(i, carry):
      base = wid * EPT + i * CH
      pltpu.sync_copy(dst_hbm.at[pl.ds(base, CH)], dst_v)
      pltpu.sync_copy(ones_v, acc.at[dst_v], add=True)
      return carry

    lax.fori_loop(0, NCH, body, 0)
    plsc.subcore_barrier()
    for j in range(RPT // CH):
      r = sid * RPT + j * CH
      pltpu.sync_copy(acc.at[pl.ds(r, CH)], buf_v)
      pltpu.sync_copy(buf_v, out_hbm.at[cid, pl.ds(r, CH)])

  return k(dst_p, ones16, zeros16)


def _seg_sum_call(F):
  mesh = plsc.VectorSubcoreMesh(**_MESH)

  @functools.partial(
      pl.kernel,
      mesh=mesh,
      out_type=jax.ShapeDtypeStruct((2, N_ACC, F), jnp.float32),
      scratch_types=[
          pltpu.VMEM((CH,), jnp.int32),
          pltpu.VMEM((CH,), jnp.int32),
          pltpu.VMEM((CH, F), jnp.float32),
          pltpu.VMEM_SHARED((N_ACC, F), jnp.float32),
          pltpu.SemaphoreType.DMA,
      ],
      compiler_params=_SC_PARAMS,
  )
  def k(y_hbm, src_hbm, dst_hbm, zeros_hbm, out_hbm, src_v, dst_v, rows_v, acc, sem):
    cid = lax.axis_index("c")
    sid = lax.axis_index("s")
    wid = sid * 2 + cid
    pltpu.sync_copy(zeros_hbm, rows_v)
    for j in range(RPT // CH):
      pltpu.sync_copy(rows_v, acc.at[pl.ds(sid * RPT + j * CH, CH)])
    plsc.subcore_barrier()

    def body(i, carry):
      base = wid * EPT + i * CH
      pltpu.sync_copy(src_hbm.at[pl.ds(base, CH)], src_v)
      pltpu.sync_copy(dst_hbm.at[pl.ds(base, CH)], dst_v)
      pltpu.async_copy(y_hbm.at[src_v], rows_v, sem).wait()
      pltpu.sync_copy(rows_v, acc.at[dst_v], add=True)
      return carry

    lax.fori_loop(0, NCH, body, 0)
    plsc.subcore_barrier()
    for j in range(RPT // CH):
      r = sid * RPT + j * CH
      pltpu.sync_copy(acc.at[pl.ds(r, CH)], rows_v)
      pltpu.sync_copy(rows_v, out_hbm.at[cid, pl.ds(r, CH)])

  return k


_seg_sum_128 = _seg_sum_call(HID)
_seg_sum_64 = _seg_sum_call(C)

RB = 256
GRID = N_ACC // RB


def _tc1_call(x_p, degp, W1):
  def body(x_ref, d_ref, w_ref, y_ref, dinv_ref):
    d = d_ref[0, :, 0:1] + d_ref[1, :, 0:1] + 1.0
    dinv = lax.rsqrt(d)
    xw = jnp.dot(x_ref[...], w_ref[...], preferred_element_type=jnp.float32)
    y_ref[...] = xw * dinv
    dinv_ref[...] = dinv

  return pl.pallas_call(
      body,
      grid=(GRID,),
      in_specs=[
          pl.BlockSpec((RB, F_IN), lambda i: (i, 0)),
          pl.BlockSpec((2, RB, DEG_W), lambda i: (0, i, 0)),
          pl.BlockSpec((F_IN, HID), lambda i: (0, 0)),
      ],
      out_specs=[
          pl.BlockSpec((RB, HID), lambda i: (i, 0)),
          pl.BlockSpec((RB, 1), lambda i: (i, 0)),
      ],
      out_shape=[
          jax.ShapeDtypeStruct((N_ACC, HID), jnp.float32),
          jax.ShapeDtypeStruct((N_ACC, 1), jnp.float32),
      ],
  )(x_p, degp, W1)


def _tc_mid_call(p, y, dinv, b, W, F_in, F_out, act):
  def body(p_ref, y_ref, dinv_ref, b_ref, w_ref, o_ref):
    dinv = dinv_ref[...]
    t = dinv * (p_ref[0] + p_ref[1] + y_ref[...]) + b_ref[...]
    if act == "relu":
      h = jnp.maximum(t, 0.0)
    else:
      m = jnp.max(t, axis=1, keepdims=True)
      ex = jnp.exp(t - m)
      h = ex / jnp.sum(ex, axis=1, keepdims=True)
    o_ref[...] = jnp.dot(h, w_ref[...], preferred_element_type=jnp.float32) * dinv

  return pl.pallas_call(
      body,
      grid=(GRID,),
      in_specs=[
          pl.BlockSpec((2, RB, F_in), lambda i: (0, i, 0)),
          pl.BlockSpec((RB, F_in), lambda i: (i, 0)),
          pl.BlockSpec((RB, 1), lambda i: (i, 0)),
          pl.BlockSpec((1, F_in), lambda i: (0, 0)),
          pl.BlockSpec((F_in, F_out), lambda i: (0, 0)),
      ],
      out_specs=pl.BlockSpec((RB, F_out), lambda i: (i, 0)),
      out_shape=jax.ShapeDtypeStruct((N_ACC, F_out), jnp.float32),
  )(p, y, dinv, b, W)


def _tc_out_call(p, y, dinv, b):
  def body(p_ref, y_ref, dinv_ref, b_ref, o_ref):
    o_ref[...] = dinv_ref[...] * (p_ref[0] + p_ref[1] + y_ref[...]) + b_ref[...]

  return pl.pallas_call(
      body,
      grid=(GRID,),
      in_specs=[
          pl.BlockSpec((2, RB, C), lambda i: (0, i, 0)),
          pl.BlockSpec((RB, C), lambda i: (i, 0)),
          pl.BlockSpec((RB, 1), lambda i: (i, 0)),
          pl.BlockSpec((1, C), lambda i: (0, 0)),
      ],
      out_specs=pl.BlockSpec((RB, C), lambda i: (i, 0)),
      out_shape=jax.ShapeDtypeStruct((N_ACC, C), jnp.float32),
  )(p, y, dinv, b)


def kernel(x, edge_index, W1, b1, W2, b2, W3, b3):
  src = edge_index[0]
  dst = edge_index[1]
  pad_e = E_PAD - E
  padv = jnp.full((pad_e,), N, jnp.int32)
  src_p = jnp.concatenate([src, padv])
  dst_p = jnp.concatenate([dst, padv])
  x_p = jnp.pad(x, ((0, N_ACC - N), (0, 0)))

  ones16 = jnp.ones((CH, DEG_W), jnp.float32)
  zeros16 = jnp.zeros((CH, DEG_W), jnp.float32)
  zeros128 = jnp.zeros((CH, HID), jnp.float32)
  zeros64 = jnp.zeros((CH, C), jnp.float32)

  degp = _deg_call(dst_p, ones16, zeros16)
  y1, dinv = _tc1_call(x_p, degp, W1)
  p1 = _seg_sum_128(y1, src_p, dst_p, zeros128)
  y2 = _tc_mid_call(p1, y1, dinv, b1.reshape(1, -1), W2, HID, C, "relu")
  p2 = _seg_sum_64(y2, src_p, dst_p, zeros64)
  y3 = _tc_mid_call(p2, y2, dinv, b2.reshape(1, -1), W3, C, C, "softmax")
  p3 = _seg_sum_64(y3, src_p, dst_p, zeros64)
  out = _tc_out_call(p3, y3, dinv, b3.reshape(1, -1))
  return out[:N]


# speedup vs baseline: 9.5876x; 1.0895x over previous
import functools

import jax
import jax.numpy as jnp
from jax import lax
from jax.experimental import pallas as pl
from jax.experimental.pallas import tpu as pltpu
from jax.experimental.pallas import tpu_sc as plsc

N = 10000
F_IN = 128
HID = 128
C = 64
E = 320000

N_ACC = 10240
NTILES = 32
EPT = 10240
E_PAD = NTILES * EPT
CH = 128
NCH = EPT // CH
RPT = N_ACC // 16
DEG_W = 16

_MESH = dict(core_axis_name="c", subcore_axis_name="s")
_SC_PARAMS = pltpu.CompilerParams(use_tc_tiling_on_sc=False)


def _deg_call(dst_p, ones16, zeros16):
  mesh = plsc.VectorSubcoreMesh(**_MESH)

  @functools.partial(
      pl.kernel,
      mesh=mesh,
      out_type=jax.ShapeDtypeStruct((2, N_ACC, DEG_W), jnp.float32),
      scratch_types=[
          pltpu.VMEM((NCH, CH), jnp.int32),
          pltpu.VMEM((CH, DEG_W), jnp.float32),
          pltpu.VMEM((CH, DEG_W), jnp.float32),
          pltpu.VMEM_SHARED((N_ACC, DEG_W), jnp.float32),
      ],
      compiler_params=_SC_PARAMS,
  )
  def k(dst_hbm, ones_hbm, zeros_hbm, out_hbm, dst_a, ones_v, buf_v, acc):
    cid = lax.axis_index("c")
    sid = lax.axis_index("s")
    wid = sid * 2 + cid
    pltpu.sync_copy(ones_hbm, ones_v)
    pltpu.sync_copy(zeros_hbm, buf_v)
    for j in range(RPT // CH):
      pltpu.sync_copy(buf_v, acc.at[pl.ds(sid * RPT + j * CH, CH)])
    pltpu.sync_copy(dst_hbm.at[pl.ds(wid * NCH, NCH)], dst_a)
    plsc.subcore_barrier()

    def body(i, carry):
      pltpu.sync_copy(ones_v, acc.at[dst_a.at[i]], add=True)
      return carry

    lax.fori_loop(0, NCH, body, 0)
    plsc.subcore_barrier()
    for j in range(RPT // CH):
      r = sid * RPT + j * CH
      pltpu.sync_copy(acc.at[pl.ds(r, CH)], buf_v)
      pltpu.sync_copy(buf_v, out_hbm.at[cid, pl.ds(r, CH)])

  return k(dst_p, ones16, zeros16)


def _seg_sum_call(F):
  mesh = plsc.VectorSubcoreMesh(**_MESH)

  @functools.partial(
      pl.kernel,
      mesh=mesh,
      out_type=jax.ShapeDtypeStruct((2, N_ACC, F), jnp.float32),
      scratch_types=[
          pltpu.VMEM((NCH, CH), jnp.int32),
          pltpu.VMEM((NCH, CH), jnp.int32),
          pltpu.VMEM((CH, F), jnp.float32),
          pltpu.VMEM((CH, F), jnp.float32),
          pltpu.VMEM_SHARED((N_ACC, F), jnp.float32),
          pltpu.SemaphoreType.DMA,
          pltpu.SemaphoreType.DMA,
      ],
      compiler_params=_SC_PARAMS,
  )
  def k(y_hbm, src_hbm, dst_hbm, zeros_hbm, out_hbm, src_a, dst_a, rows_a, rows_b, acc, sem_a, sem_b):
    cid = lax.axis_index("c")
    sid = lax.axis_index("s")
    wid = sid * 2 + cid
    pltpu.sync_copy(zeros_hbm, rows_a)
    for j in range(RPT // CH):
      pltpu.sync_copy(rows_a, acc.at[pl.ds(sid * RPT + j * CH, CH)])
    pltpu.sync_copy(src_hbm.at[pl.ds(wid * NCH, NCH)], src_a)
    pltpu.sync_copy(dst_hbm.at[pl.ds(wid * NCH, NCH)], dst_a)
    plsc.subcore_barrier()

    pltpu.async_copy(y_hbm.at[src_a.at[0]], rows_a, sem_a)
    half = NCH // 2

    def body(i, carry):
      pltpu.async_copy(y_hbm.at[src_a.at[2 * i + 1]], rows_b, sem_b)
      pltpu.make_async_copy(y_hbm.at[pl.ds(0, CH)], rows_a, sem_a).wait()
      pltpu.sync_copy(rows_a, acc.at[dst_a.at[2 * i]], add=True)

      @pl.when(i < half - 1)
      def _():
        pltpu.async_copy(y_hbm.at[src_a.at[2 * i + 2]], rows_a, sem_a)

      pltpu.make_async_copy(y_hbm.at[pl.ds(0, CH)], rows_b, sem_b).wait()
      pltpu.sync_copy(rows_b, acc.at[dst_a.at[2 * i + 1]], add=True)
      return carry

    lax.fori_loop(0, half, body, 0)
    plsc.subcore_barrier()
    for j in range(RPT // CH):
      r = sid * RPT + j * CH
      pltpu.sync_copy(acc.at[pl.ds(r, CH)], rows_a)
      pltpu.sync_copy(rows_a, out_hbm.at[cid, pl.ds(r, CH)])

  return k


_seg_sum_64 = _seg_sum_call(C)

RB = 256
GRID = N_ACC // RB


def _tc1_call(x_p, degp, W1):
  def body(x_ref, d_ref, w_ref, ya_ref, yb_ref, dinv_ref):
    d = d_ref[0, :, 0:1] + d_ref[1, :, 0:1] + 1.0
    dinv = lax.rsqrt(d)
    xw = jnp.dot(x_ref[...], w_ref[...], preferred_element_type=jnp.float32)
    y = xw * dinv
    ya_ref[...] = y[:, :C]
    yb_ref[...] = y[:, C:]
    dinv_ref[...] = dinv

  return pl.pallas_call(
      body,
      grid=(GRID,),
      in_specs=[
          pl.BlockSpec((RB, F_IN), lambda i: (i, 0)),
          pl.BlockSpec((2, RB, DEG_W), lambda i: (0, i, 0)),
          pl.BlockSpec((F_IN, HID), lambda i: (0, 0)),
      ],
      out_specs=[
          pl.BlockSpec((RB, C), lambda i: (i, 0)),
          pl.BlockSpec((RB, C), lambda i: (i, 0)),
          pl.BlockSpec((RB, 1), lambda i: (i, 0)),
      ],
      out_shape=[
          jax.ShapeDtypeStruct((N_ACC, C), jnp.float32),
          jax.ShapeDtypeStruct((N_ACC, C), jnp.float32),
          jax.ShapeDtypeStruct((N_ACC, 1), jnp.float32),
      ],
  )(x_p, degp, W1)


def _tc2_call(pa, pb, ya, yb, dinv, b, W):
  def body(pa_ref, pb_ref, ya_ref, yb_ref, dinv_ref, b_ref, w_ref, o_ref):
    dinv = dinv_ref[...]
    ta = dinv * (pa_ref[0] + pa_ref[1] + ya_ref[...]) + b_ref[:, :C]
    tb = dinv * (pb_ref[0] + pb_ref[1] + yb_ref[...]) + b_ref[:, C:]
    h = jnp.maximum(jnp.concatenate([ta, tb], axis=1), 0.0)
    o_ref[...] = jnp.dot(h, w_ref[...], preferred_element_type=jnp.float32) * dinv

  return pl.pallas_call(
      body,
      grid=(GRID,),
      in_specs=[
          pl.BlockSpec((2, RB, C), lambda i: (0, i, 0)),
          pl.BlockSpec((2, RB, C), lambda i: (0, i, 0)),
          pl.BlockSpec((RB, C), lambda i: (i, 0)),
          pl.BlockSpec((RB, C), lambda i: (i, 0)),
          pl.BlockSpec((RB, 1), lambda i: (i, 0)),
          pl.BlockSpec((1, HID), lambda i: (0, 0)),
          pl.BlockSpec((HID, C), lambda i: (0, 0)),
      ],
      out_specs=pl.BlockSpec((RB, C), lambda i: (i, 0)),
      out_shape=jax.ShapeDtypeStruct((N_ACC, C), jnp.float32),
  )(pa, pb, ya, yb, dinv, b, W)


def _tc_mid_call(p, y, dinv, b, W, F_in, F_out, act):
  def body(p_ref, y_ref, dinv_ref, b_ref, w_ref, o_ref):
    dinv = dinv_ref[...]
    t = dinv * (p_ref[0] + p_ref[1] + y_ref[...]) + b_ref[...]
    if act == "relu":
      h = jnp.maximum(t, 0.0)
    else:
      m = jnp.max(t, axis=1, keepdims=True)
      ex = jnp.exp(t - m)
      h = ex / jnp.sum(ex, axis=1, keepdims=True)
    o_ref[...] = jnp.dot(h, w_ref[...], preferred_element_type=jnp.float32) * dinv

  return pl.pallas_call(
      body,
      grid=(GRID,),
      in_specs=[
          pl.BlockSpec((2, RB, F_in), lambda i: (0, i, 0)),
          pl.BlockSpec((RB, F_in), lambda i: (i, 0)),
          pl.BlockSpec((RB, 1), lambda i: (i, 0)),
          pl.BlockSpec((1, F_in), lambda i: (0, 0)),
          pl.BlockSpec((F_in, F_out), lambda i: (0, 0)),
      ],
      out_specs=pl.BlockSpec((RB, F_out), lambda i: (i, 0)),
      out_shape=jax.ShapeDtypeStruct((N_ACC, F_out), jnp.float32),
  )(p, y, dinv, b, W)


def _tc_out_call(p, y, dinv, b):
  def body(p_ref, y_ref, dinv_ref, b_ref, o_ref):
    o_ref[...] = dinv_ref[...] * (p_ref[0] + p_ref[1] + y_ref[...]) + b_ref[...]

  return pl.pallas_call(
      body,
      grid=(GRID,),
      in_specs=[
          pl.BlockSpec((2, RB, C), lambda i: (0, i, 0)),
          pl.BlockSpec((RB, C), lambda i: (i, 0)),
          pl.BlockSpec((RB, 1), lambda i: (i, 0)),
          pl.BlockSpec((1, C), lambda i: (0, 0)),
      ],
      out_specs=pl.BlockSpec((RB, C), lambda i: (i, 0)),
      out_shape=jax.ShapeDtypeStruct((N_ACC, C), jnp.float32),
  )(p, y, dinv, b)


def kernel(x, edge_index, W1, b1, W2, b2, W3, b3):
  src = edge_index[0]
  dst = edge_index[1]
  pad_e = E_PAD - E
  padv = jnp.full((pad_e,), N, jnp.int32)
  src_p = jnp.concatenate([src, padv]).reshape(E_PAD // CH, CH)
  dst_p = jnp.concatenate([dst, padv]).reshape(E_PAD // CH, CH)
  x_p = jnp.pad(x, ((0, N_ACC - N), (0, 0)))

  ones16 = jnp.ones((CH, DEG_W), jnp.float32)
  zeros16 = jnp.zeros((CH, DEG_W), jnp.float32)
  zeros64 = jnp.zeros((CH, C), jnp.float32)

  degp = _deg_call(dst_p, ones16, zeros16)
  y1a, y1b, dinv = _tc1_call(x_p, degp, W1)
  p1a = _seg_sum_64(y1a, src_p, dst_p, zeros64)
  p1b = _seg_sum_64(y1b, src_p, dst_p, zeros64)
  y2 = _tc2_call(p1a, p1b, y1a, y1b, dinv, b1.reshape(1, -1), W2)
  p2 = _seg_sum_64(y2, src_p, dst_p, zeros64)
  y3 = _tc_mid_call(p2, y2, dinv, b2.reshape(1, -1), W3, C, C, "softmax")
  p3 = _seg_sum_64(y3, src_p, dst_p, zeros64)
  out = _tc_out_call(p3, y3, dinv, b3.reshape(1, -1))
  return out[:N]
